# baseline (speedup 1.0000x reference)
import functools

import jax
import jax.numpy as jnp
from jax import lax
from jax.experimental import pallas as pl
from jax.experimental.pallas import tpu as pltpu
from jax.experimental.pallas import tpu_sc as plsc

_F32 = jnp.float32


def _pack_cols(q):
    outs = []
    for k in range(q.shape[1] // 32):
        ba = lax.bitcast_convert_type(q[:, 32 * k:32 * k + 16], jnp.int32)
        bb = lax.bitcast_convert_type(q[:, 32 * k + 16:32 * k + 32], jnp.int32)
        ra = ba + 0x7FFF + ((ba >> 16) & 1)
        rb = bb + 0x7FFF + ((bb >> 16) & 1)
        outs.append(((ra >> 16) & 0xFFFF) | (rb & jnp.int32(-65536)))
    return jnp.concatenate(outs, axis=1)


def _emit_tables(x, w_ref, b_ref, ot0, ot1):
    d = x.shape[1]
    q0 = jnp.dot(x, w_ref[0:d, :], preferred_element_type=_F32) + b_ref[0:1, :]
    q1 = jnp.dot(x, w_ref[d:2 * d, :], preferred_element_type=_F32)
    for ref, q in ((ot0, q0), (ot1, q1)):
        ref[:, 0:d // 2] = _pack_cols(q[:, 0:d])
        ref[:, d // 2:d] = _pack_cols(q[:, d:2 * d])


def _mm0_body(x_ref, wa_ref, ba_ref, wb_ref, bb_ref,
              t0a, t1a, t0b, t1b):
    x = x_ref[...]
    _emit_tables(x, wa_ref, ba_ref, t0a, t1a)
    _emit_tables(x, wb_ref, bb_ref, t0b, t1b)


def _merged_x(acca_ref, cnta_ref, accb_ref, cntb_ref):
    ca = cnta_ref[0, :, 0:1] + cnta_ref[1, :, 0:1]
    cb = cntb_ref[0, :, 0:1] + cntb_ref[1, :, 0:1]
    sa = acca_ref[0] + acca_ref[1]
    sb = accb_ref[0] + accb_ref[1]
    return sa / jnp.maximum(ca, 1.0) + sb / jnp.maximum(cb, 1.0)


def _mm1_body(acca_ref, cnta_ref, accb_ref, cntb_ref,
              wa_ref, ba_ref, wb_ref, bb_ref,
              t0a, t1a, t0b, t1b):
    x = _merged_x(acca_ref, cnta_ref, accb_ref, cntb_ref)
    _emit_tables(x, wa_ref, ba_ref, t0a, t1a)
    _emit_tables(x, wb_ref, bb_ref, t0b, t1b)


def _final_body(acca_ref, cnta_ref, accb_ref, cntb_ref, out_ref):
    out_ref[...] = _merged_x(acca_ref, cnta_ref, accb_ref, cntb_ref)


def _tc_specs(n, d, bn):
    grid = (n // bn,)
    x_spec = pl.BlockSpec((bn, d), lambda i: (i, 0))
    w_spec = pl.BlockSpec((2 * d, 2 * d), lambda i: (0, 0))
    b_spec = pl.BlockSpec((1, 2 * d), lambda i: (0, 0))
    acc_spec = pl.BlockSpec((2, bn, d), lambda i: (0, i, 0))
    cnt_spec = pl.BlockSpec((2, bn, 16), lambda i: (0, i, 0))
    tab_shape = [jax.ShapeDtypeStruct((n, d), jnp.int32)] * 4
    tab_specs = [pl.BlockSpec((bn, d), lambda i: (i, 0))] * 4
    return grid, x_spec, w_spec, b_spec, acc_spec, cnt_spec, tab_shape, tab_specs


def _mm_layer0(x, wa, ba, wb, bb, bn=1000):
    n, d = x.shape
    grid, x_spec, w_spec, b_spec, _, _, tab_shape, tab_specs = _tc_specs(n, d, bn)
    return pl.pallas_call(
        _mm0_body, grid=grid,
        in_specs=[x_spec, w_spec, b_spec, w_spec, b_spec],
        out_specs=tab_specs, out_shape=tab_shape,
    )(x, wa, ba, wb, bb)


def _mm_layer1(acca, cnta, accb, cntb, wa, ba, wb, bb, bn=1000):
    n, d = acca.shape[1], acca.shape[2]
    grid, _, w_spec, b_spec, acc_spec, cnt_spec, tab_shape, tab_specs = _tc_specs(n, d, bn)
    return pl.pallas_call(
        _mm1_body, grid=grid,
        in_specs=[acc_spec, cnt_spec, acc_spec, cnt_spec,
                  w_spec, b_spec, w_spec, b_spec],
        out_specs=tab_specs, out_shape=tab_shape,
    )(acca, cnta, accb, cntb, wa, ba, wb, bb)


def _merge_final(acca, cnta, accb, cntb, bn=1000):
    n, d = acca.shape[1], acca.shape[2]
    grid, x_spec, _, _, acc_spec, cnt_spec, _, _ = _tc_specs(n, d, bn)
    return pl.pallas_call(
        _final_body, grid=grid,
        in_specs=[acc_spec, cnt_spec, acc_spec, cnt_spec],
        out_specs=x_spec, out_shape=jax.ShapeDtypeStruct((n, d), _F32),
    )(acca, cnta, accb, cntb)


def _sc_geometry(n, d, ns):
    dr = 80
    rows_per_tile = -(-(-(-n // ns)) // dr) * dr
    npad = rows_per_tile * ns
    return rows_per_tile, npad, dr


@functools.lru_cache(maxsize=None)
def _make_sc_count(n, d, g):
    info = plsc.get_sparse_core_info()
    nc, ns = info.num_cores, info.num_subcores
    nw = nc * ns
    ch = 128
    nchunks = g // ch
    assert nchunks * ch == g
    rows_per_tile, npad, dr = _sc_geometry(n, d, ns)
    cw = d

    mesh = plsc.VectorSubcoreMesh(core_axis_name="c", subcore_axis_name="s")
    out_type = [jax.ShapeDtypeStruct((nc * npad, cw), _F32)]
    scratch = [
        pltpu.VMEM((ch,), jnp.int32),
        pltpu.VMEM((ch,), jnp.int32),
        pltpu.VMEM((ch, cw), _F32),
        pltpu.VMEM_SHARED((npad, cw), _F32),
        pltpu.SemaphoreType.DMA,
        pltpu.SemaphoreType.DMA,
    ]

    def body(i0h, i1h, cnt_out, idx0, idx1, ones_v, cntacc, isem, ssem):
        cid = lax.axis_index("c")
        sid = lax.axis_index("s")
        wid = sid * nc + cid
        row0 = sid * rows_per_tile

        def _fill(val):
            def _row(r, _):
                for k in range(cw // 16):
                    ones_v[r, pl.ds(k * 16, 16)] = jnp.full((16,), val, _F32)
                return 0
            lax.fori_loop(0, ch, _row, 0)

        _fill(0.0)
        for c in range(rows_per_tile // dr):
            pltpu.sync_copy(ones_v.at[pl.ds(0, dr)],
                            cntacc.at[pl.ds(row0 + c * dr, dr)])
        _fill(1.0)
        plsc.subcore_barrier()

        base_chunks = nchunks // nw
        rem = nchunks % nw
        my_chunks = base_chunks + jnp.where(wid < rem, 1, 0)

        def _chunk(j, _):
            base = (j * nw + wid) * ch
            pltpu.async_copy(i0h.at[pl.ds(base, ch)], idx0, isem)
            pltpu.async_copy(i1h.at[pl.ds(base, ch)], idx1, isem)
            pltpu.make_async_copy(i0h.at[pl.ds(base, ch)], idx0, isem).wait()
            pltpu.make_async_copy(i1h.at[pl.ds(base, ch)], idx1, isem).wait()
            pltpu.async_copy(ones_v, cntacc.at[idx0], ssem, add=True)
            pltpu.async_copy(ones_v, cntacc.at[idx1], ssem, add=True)
            pltpu.make_async_copy(ones_v, cntacc.at[idx0], ssem).wait()
            pltpu.make_async_copy(ones_v, cntacc.at[idx1], ssem).wait()
            return 0
        lax.fori_loop(0, my_chunks, _chunk, 0)

        plsc.subcore_barrier()
        pltpu.sync_copy(cntacc.at[pl.ds(row0, rows_per_tile)],
                        cnt_out.at[pl.ds(cid * npad + row0, rows_per_tile)])

    return pl.kernel(body, mesh=mesh, out_type=out_type, scratch_types=scratch)


@functools.lru_cache(maxsize=None)
def _make_sc_pass(n, d, g):
    info = plsc.get_sparse_core_info()
    nc, ns = info.num_cores, info.num_subcores
    nw = nc * ns
    ch = 40
    nchunks = g // ch
    assert nchunks * ch == g and nchunks % nw == 0
    per_tile = nchunks // nw
    npairs = (per_tile - 1) // 2
    assert per_tile == 2 * npairs + 1
    rows_per_tile, npad, dr = _sc_geometry(n, d, ns)
    assert dr % ch == 0
    nseg = d // 16

    mesh = plsc.VectorSubcoreMesh(core_axis_name="c", subcore_axis_name="s")

    out_type = [jax.ShapeDtypeStruct((nc * npad, d), _F32)] * 2
    ubuf = lambda: pltpu.VMEM((ch, d), jnp.int32)
    hbuf = lambda: pltpu.VMEM((ch, d), _F32)
    idxbuf = lambda: pltpu.VMEM((ch,), jnp.int32)
    scratch = (
        [idxbuf() for _ in range(8)]
        + [ubuf() for _ in range(4)]
        + [hbuf() for _ in range(4)]
        + [pltpu.VMEM_SHARED((npad, d), _F32),
           pltpu.SemaphoreType.DMA, pltpu.SemaphoreType.DMA,
           pltpu.SemaphoreType.DMA, pltpu.SemaphoreType.DMA,
           pltpu.SemaphoreType.DMA, pltpu.SemaphoreType.DMA]
    )

    def body(tfa0, tfa1, tfb0, tfb1, ia0h, ia1h, ib0h, ib1h,
             acc_out_a, acc_out_b,
             i0s0, i1s0, i0s1, i1s1, c0s0, c1s0, c0s1, c1s1,
             u0s0, u1s0, u0s1, u1s1,
             h0s0, h1s0, h0s1, h1s1,
             accum, sem0, sem1, ssem0, ssem1, isem0, isem1):
        sets = ((i0s0, i1s0, c0s0, c1s0, u0s0, u1s0, h0s0, h1s0, sem0, ssem0, isem0),
                (i0s1, i1s1, c0s1, c1s1, u0s1, u1s1, h0s1, h1s1, sem1, ssem1, isem1))
        cid = lax.axis_index("c")
        sid = lax.axis_index("s")
        wid = sid * nc + cid
        row0 = sid * rows_per_tile

        def _run_pass(ta0, ta1, i0h, i1h, acc_out):
            def _zrow(r, _):
                for k in range(nseg):
                    h0s0[r, pl.ds(k * 16, 16)] = jnp.zeros((16,), _F32)
                return 0
            lax.fori_loop(0, ch, _zrow, 0)
            for c in range(rows_per_tile // ch):
                pltpu.async_copy(h0s0, accum.at[pl.ds(row0 + c * ch, ch)], ssem0)
            for c in range(rows_per_tile // ch):
                pltpu.make_async_copy(
                    h0s0, accum.at[pl.ds(row0 + c * ch, ch)], ssem0).wait()
            plsc.subcore_barrier()

            def _idx_base(j):
                return (j * nw + wid) * ch

            def _prefetch_idx(s, j):
                i0b, i1b = sets[s][0], sets[s][1]
                isem = sets[s][10]
                base = _idx_base(j)
                pltpu.async_copy(i0h.at[pl.ds(base, ch)], i0b, isem)
                pltpu.async_copy(i1h.at[pl.ds(base, ch)], i1b, isem)

            def _issue(s, j):
                i0b, i1b, _, _, u0, u1, _, _, sem, _, isem = sets[s]
                base = _idx_base(j)
                pltpu.make_async_copy(i0h.at[pl.ds(base, ch)], i0b, isem).wait()
                pltpu.make_async_copy(i1h.at[pl.ds(base, ch)], i1b, isem).wait()
                pltpu.async_copy(ta0.at[i0b], u0, sem)
                pltpu.async_copy(ta1.at[i1b], u1, sem)

            def _save_idx(s):
                i0b, i1b, c0b, c1b = sets[s][0], sets[s][1], sets[s][2], sets[s][3]
                for off in (0, 16, ch - 16):
                    sl = pl.ds(off, 16)
                    c0b[sl] = i0b[sl]
                    c1b[sl] = i1b[sl]

            def _wait_gathers(s):
                i0b, i1b, _, _, u0, u1, _, _, sem, _, _ = sets[s]
                pltpu.make_async_copy(ta0.at[i0b], u0, sem).wait()
                pltpu.make_async_copy(ta1.at[i1b], u1, sem).wait()

            def _consume(s):
                _, _, c0b, c1b, u0, u1, h0, h1, sem, ssem, _ = sets[s]

                def _row(r2, _):
                    for sub in (0, 1):
                        r = 2 * r2 + sub
                        for half, hb in ((0, h0), (1, h1)):
                            for k in range(d // 32):
                                w0 = u0[r, pl.ds(half * (d // 2) + k * 16, 16)]
                                w1 = u1[r, pl.ds(half * (d // 2) + k * 16, 16)]
                                lo0 = lax.bitcast_convert_type(w0 << 16, _F32)
                                lo1 = lax.bitcast_convert_type(w1 << 16, _F32)
                                hi0 = lax.bitcast_convert_type(
                                    w0 & jnp.int32(-65536), _F32)
                                hi1 = lax.bitcast_convert_type(
                                    w1 & jnp.int32(-65536), _F32)
                                hb[r, pl.ds(k * 32, 16)] = jnp.maximum(
                                    lo0 + lo1, 0.0)
                                hb[r, pl.ds(k * 32 + 16, 16)] = jnp.maximum(
                                    hi0 + hi1, 0.0)
                    return 0
                lax.fori_loop(0, ch // 2, _row, 0)

                pltpu.async_copy(h0, accum.at[c0b], ssem, add=True)
                pltpu.async_copy(h1, accum.at[c1b], ssem, add=True)

            def _wait_scatter(s):
                _, _, c0b, c1b, _, _, h0, h1, _, ssem, _ = sets[s]
                pltpu.make_async_copy(h0, accum.at[c0b], ssem).wait()
                pltpu.make_async_copy(h1, accum.at[c1b], ssem).wait()

            _prefetch_idx(0, 0)
            _prefetch_idx(1, 1)
            _issue(0, 0)
            _issue(1, 1)

            def _pair(p, _):
                @pl.when(p > 0)
                def _():
                    _wait_scatter(0)
                    _wait_scatter(1)
                _wait_gathers(0)
                _save_idx(0)
                _prefetch_idx(0, 2 * p + 2)
                _consume(0)
                _issue(0, 2 * p + 2)

                _wait_gathers(1)
                _save_idx(1)

                @pl.when(p < npairs - 1)
                def _():
                    _prefetch_idx(1, 2 * p + 3)
                _consume(1)

                @pl.when(p < npairs - 1)
                def _():
                    _issue(1, 2 * p + 3)
                return 0
            lax.fori_loop(0, npairs, _pair, 0)
            _wait_scatter(0)
            _wait_scatter(1)
            _wait_gathers(0)
            _save_idx(0)
            _consume(0)
            _wait_scatter(0)

            plsc.subcore_barrier()
            pltpu.sync_copy(accum.at[pl.ds(row0, rows_per_tile)],
                            acc_out.at[pl.ds(cid * npad + row0, rows_per_tile)])

        _run_pass(tfa0, tfa1, ia0h, ia1h, acc_out_a)
        plsc.subcore_barrier()
        _run_pass(tfb0, tfb1, ib0h, ib1h, acc_out_b)

    return pl.kernel(body, mesh=mesh, out_type=out_type, scratch_types=scratch), npad


def kernel(inputs, grounding_indices_0, grounding_indices_1,
           W_f0_t0, b_f0_t0, W_f0_t1, b_f0_t1,
           W_f1_t0, b_f1_t0, W_f1_t1, b_f1_t1,
           attention_weights):
    x = inputs[0]
    n, d = x.shape
    g = grounding_indices_0.shape[0]

    ia0 = grounding_indices_0[:, 0].astype(jnp.int32)
    ia1 = grounding_indices_0[:, 1].astype(jnp.int32)
    ib0 = grounding_indices_1[:, 0].astype(jnp.int32)
    ib1 = grounding_indices_1[:, 1].astype(jnp.int32)

    ba0 = b_f0_t0.reshape(1, -1)
    ba1 = b_f0_t1.reshape(1, -1)
    bb0 = b_f1_t0.reshape(1, -1)
    bb1 = b_f1_t1.reshape(1, -1)

    sc_pass, npad = _make_sc_pass(n, d, g)
    sc_count = _make_sc_count(n, d, g)

    cnta = sc_count(ia0, ia1)[0].reshape(2, npad, d)[:, :n, 0:16]
    cntb = sc_count(ib0, ib1)[0].reshape(2, npad, d)[:, :n, 0:16]

    tabs = _mm_layer0(x, W_f0_t0, ba0, W_f1_t0, bb0)
    acca, accb = sc_pass(tabs[0], tabs[1], tabs[2], tabs[3],
                         ia0, ia1, ib0, ib1)
    acca = acca.reshape(2, npad, d)[:, :n]
    accb = accb.reshape(2, npad, d)[:, :n]

    tabs1 = _mm_layer1(acca, cnta, accb, cntb, W_f0_t1, ba1, W_f1_t1, bb1)
    acca1, accb1 = sc_pass(tabs1[0], tabs1[1], tabs1[2], tabs1[3],
                           ia0, ia1, ib0, ib1)
    acca1 = acca1.reshape(2, npad, d)[:, :n]
    accb1 = accb1.reshape(2, npad, d)[:, :n]

    out = _merge_final(acca1, cnta, accb1, cntb)
    return out[None]


# speedup vs baseline: 1.0077x; 1.0077x over previous
import functools

import jax
import jax.numpy as jnp
from jax import lax
from jax.experimental import pallas as pl
from jax.experimental.pallas import tpu as pltpu
from jax.experimental.pallas import tpu_sc as plsc

_F32 = jnp.float32


def _pack_cols(q):
    outs = []
    for k in range(q.shape[1] // 32):
        ba = lax.bitcast_convert_type(q[:, 32 * k:32 * k + 16], jnp.int32)
        bb = lax.bitcast_convert_type(q[:, 32 * k + 16:32 * k + 32], jnp.int32)
        ra = ba + 0x7FFF + ((ba >> 16) & 1)
        rb = bb + 0x7FFF + ((bb >> 16) & 1)
        outs.append(((ra >> 16) & 0xFFFF) | (rb & jnp.int32(-65536)))
    return jnp.concatenate(outs, axis=1)


def _emit_tables(x, w_ref, b_ref, ot0, ot1):
    d = x.shape[1]
    q0 = jnp.dot(x, w_ref[0:d, :], preferred_element_type=_F32) + b_ref[0:1, :]
    q1 = jnp.dot(x, w_ref[d:2 * d, :], preferred_element_type=_F32)
    for ref, q in ((ot0, q0), (ot1, q1)):
        ref[:, 0:d // 2] = _pack_cols(q[:, 0:d])
        ref[:, d // 2:d] = _pack_cols(q[:, d:2 * d])


def _mm0_body(x_ref, wa_ref, ba_ref, wb_ref, bb_ref,
              t0a, t1a, t0b, t1b):
    x = x_ref[...]
    _emit_tables(x, wa_ref, ba_ref, t0a, t1a)
    _emit_tables(x, wb_ref, bb_ref, t0b, t1b)


def _merged_x(acca_ref, cnta_ref, accb_ref, cntb_ref):
    ca = cnta_ref[0, :, 0:1] + cnta_ref[1, :, 0:1]
    cb = cntb_ref[0, :, 0:1] + cntb_ref[1, :, 0:1]
    sa = acca_ref[0] + acca_ref[1]
    sb = accb_ref[0] + accb_ref[1]
    return sa / jnp.maximum(ca, 1.0) + sb / jnp.maximum(cb, 1.0)


def _mm1_body(acca_ref, cnta_ref, accb_ref, cntb_ref,
              wa_ref, ba_ref, wb_ref, bb_ref,
              t0a, t1a, t0b, t1b):
    x = _merged_x(acca_ref, cnta_ref, accb_ref, cntb_ref)
    _emit_tables(x, wa_ref, ba_ref, t0a, t1a)
    _emit_tables(x, wb_ref, bb_ref, t0b, t1b)


def _final_body(acca_ref, cnta_ref, accb_ref, cntb_ref, out_ref):
    out_ref[...] = _merged_x(acca_ref, cnta_ref, accb_ref, cntb_ref)


def _tc_specs(n, d, bn):
    grid = (n // bn,)
    x_spec = pl.BlockSpec((bn, d), lambda i: (i, 0))
    w_spec = pl.BlockSpec((2 * d, 2 * d), lambda i: (0, 0))
    b_spec = pl.BlockSpec((1, 2 * d), lambda i: (0, 0))
    acc_spec = pl.BlockSpec((2, bn, d), lambda i: (0, i, 0))
    cnt_spec = pl.BlockSpec((2, bn, 16), lambda i: (0, i, 0))
    tab_shape = [jax.ShapeDtypeStruct((n, d), jnp.int32)] * 4
    tab_specs = [pl.BlockSpec((bn, d), lambda i: (i, 0))] * 4
    return grid, x_spec, w_spec, b_spec, acc_spec, cnt_spec, tab_shape, tab_specs


def _mm_layer0(x, wa, ba, wb, bb, bn=1000):
    n, d = x.shape
    grid, x_spec, w_spec, b_spec, _, _, tab_shape, tab_specs = _tc_specs(n, d, bn)
    return pl.pallas_call(
        _mm0_body, grid=grid,
        in_specs=[x_spec, w_spec, b_spec, w_spec, b_spec],
        out_specs=tab_specs, out_shape=tab_shape,
    )(x, wa, ba, wb, bb)


def _mm_layer1(acca, cnta, accb, cntb, wa, ba, wb, bb, bn=1000):
    n, d = acca.shape[1], acca.shape[2]
    grid, _, w_spec, b_spec, acc_spec, cnt_spec, tab_shape, tab_specs = _tc_specs(n, d, bn)
    return pl.pallas_call(
        _mm1_body, grid=grid,
        in_specs=[acc_spec, cnt_spec, acc_spec, cnt_spec,
                  w_spec, b_spec, w_spec, b_spec],
        out_specs=tab_specs, out_shape=tab_shape,
    )(acca, cnta, accb, cntb, wa, ba, wb, bb)


def _merge_final(acca, cnta, accb, cntb, bn=1000):
    n, d = acca.shape[1], acca.shape[2]
    grid, x_spec, _, _, acc_spec, cnt_spec, _, _ = _tc_specs(n, d, bn)
    return pl.pallas_call(
        _final_body, grid=grid,
        in_specs=[acc_spec, cnt_spec, acc_spec, cnt_spec],
        out_specs=x_spec, out_shape=jax.ShapeDtypeStruct((n, d), _F32),
    )(acca, cnta, accb, cntb)


def _sc_geometry(n, d, ns):
    dr = 80
    rows_per_tile = -(-(-(-n // ns)) // dr) * dr
    npad = rows_per_tile * ns
    return rows_per_tile, npad, dr


@functools.lru_cache(maxsize=None)
def _make_sc_count(n, d, g):
    info = plsc.get_sparse_core_info()
    nc, ns = info.num_cores, info.num_subcores
    nw = nc * ns
    ch = 128
    nchunks = g // ch
    assert nchunks * ch == g
    rows_per_tile, npad, dr = _sc_geometry(n, d, ns)
    cw = d

    mesh = plsc.VectorSubcoreMesh(core_axis_name="c", subcore_axis_name="s")
    out_type = [jax.ShapeDtypeStruct((nc * npad, cw), _F32)]
    scratch = [
        pltpu.VMEM((ch,), jnp.int32),
        pltpu.VMEM((ch,), jnp.int32),
        pltpu.VMEM((ch, cw), _F32),
        pltpu.VMEM_SHARED((npad, cw), _F32),
        pltpu.SemaphoreType.DMA,
        pltpu.SemaphoreType.DMA,
    ]

    def body(i0h, i1h, cnt_out, idx0, idx1, ones_v, cntacc, isem, ssem):
        cid = lax.axis_index("c")
        sid = lax.axis_index("s")
        wid = sid * nc + cid
        row0 = sid * rows_per_tile

        def _fill(val):
            def _row(r, _):
                for k in range(cw // 16):
                    ones_v[r, pl.ds(k * 16, 16)] = jnp.full((16,), val, _F32)
                return 0
            lax.fori_loop(0, ch, _row, 0)

        _fill(0.0)
        for c in range(rows_per_tile // dr):
            pltpu.sync_copy(ones_v.at[pl.ds(0, dr)],
                            cntacc.at[pl.ds(row0 + c * dr, dr)])
        _fill(1.0)
        plsc.subcore_barrier()

        base_chunks = nchunks // nw
        rem = nchunks % nw
        my_chunks = base_chunks + jnp.where(wid < rem, 1, 0)

        def _chunk(j, _):
            base = (j * nw + wid) * ch
            pltpu.async_copy(i0h.at[pl.ds(base, ch)], idx0, isem)
            pltpu.async_copy(i1h.at[pl.ds(base, ch)], idx1, isem)
            pltpu.make_async_copy(i0h.at[pl.ds(base, ch)], idx0, isem).wait()
            pltpu.make_async_copy(i1h.at[pl.ds(base, ch)], idx1, isem).wait()
            pltpu.async_copy(ones_v, cntacc.at[idx0], ssem, add=True)
            pltpu.async_copy(ones_v, cntacc.at[idx1], ssem, add=True)
            pltpu.make_async_copy(ones_v, cntacc.at[idx0], ssem).wait()
            pltpu.make_async_copy(ones_v, cntacc.at[idx1], ssem).wait()
            return 0
        lax.fori_loop(0, my_chunks, _chunk, 0)

        plsc.subcore_barrier()
        pltpu.sync_copy(cntacc.at[pl.ds(row0, rows_per_tile)],
                        cnt_out.at[pl.ds(cid * npad + row0, rows_per_tile)])

    return pl.kernel(body, mesh=mesh, out_type=out_type, scratch_types=scratch)


@functools.lru_cache(maxsize=None)
def _make_sc_pass(n, d, g):
    info = plsc.get_sparse_core_info()
    nc, ns = info.num_cores, info.num_subcores
    nw = nc * ns
    ch = 40
    nchunks = g // ch
    assert nchunks * ch == g and nchunks % nw == 0
    per_tile = nchunks // nw
    npairs = (per_tile - 1) // 2
    assert per_tile == 2 * npairs + 1
    rows_per_tile, npad, dr = _sc_geometry(n, d, ns)
    assert dr % ch == 0
    nseg = d // 16

    mesh = plsc.VectorSubcoreMesh(core_axis_name="c", subcore_axis_name="s")

    out_type = [jax.ShapeDtypeStruct((nc * npad, d), _F32)]
    ubuf = lambda: pltpu.VMEM((ch, d), jnp.int32)
    hbuf = lambda: pltpu.VMEM((ch, d), _F32)
    idxbuf = lambda: pltpu.VMEM((ch,), jnp.int32)
    scratch = (
        [idxbuf() for _ in range(8)]
        + [ubuf() for _ in range(4)]
        + [hbuf() for _ in range(4)]
        + [pltpu.VMEM_SHARED((npad, d), _F32),
           pltpu.SemaphoreType.DMA, pltpu.SemaphoreType.DMA,
           pltpu.SemaphoreType.DMA, pltpu.SemaphoreType.DMA,
           pltpu.SemaphoreType.DMA, pltpu.SemaphoreType.DMA]
    )

    def body(tfa0, tfa1, ia0h, ia1h, acc_out_a,
             i0s0, i1s0, i0s1, i1s1, c0s0, c1s0, c0s1, c1s1,
             u0s0, u1s0, u0s1, u1s1,
             h0s0, h1s0, h0s1, h1s1,
             accum, sem0, sem1, ssem0, ssem1, isem0, isem1):
        sets = ((i0s0, i1s0, c0s0, c1s0, u0s0, u1s0, h0s0, h1s0, sem0, ssem0, isem0),
                (i0s1, i1s1, c0s1, c1s1, u0s1, u1s1, h0s1, h1s1, sem1, ssem1, isem1))
        cid = lax.axis_index("c")
        sid = lax.axis_index("s")
        wid = sid * nc + cid
        row0 = sid * rows_per_tile

        def _run_pass(ta0, ta1, i0h, i1h, acc_out):
            def _zrow(r, _):
                for k in range(nseg):
                    h0s0[r, pl.ds(k * 16, 16)] = jnp.zeros((16,), _F32)
                return 0
            lax.fori_loop(0, ch, _zrow, 0)
            for c in range(rows_per_tile // ch):
                pltpu.async_copy(h0s0, accum.at[pl.ds(row0 + c * ch, ch)], ssem0)
            for c in range(rows_per_tile // ch):
                pltpu.make_async_copy(
                    h0s0, accum.at[pl.ds(row0 + c * ch, ch)], ssem0).wait()
            plsc.subcore_barrier()

            def _idx_base(j):
                return (j * nw + wid) * ch

            def _prefetch_idx(s, j):
                i0b, i1b = sets[s][0], sets[s][1]
                isem = sets[s][10]
                base = _idx_base(j)
                pltpu.async_copy(i0h.at[pl.ds(base, ch)], i0b, isem)
                pltpu.async_copy(i1h.at[pl.ds(base, ch)], i1b, isem)

            def _issue(s, j):
                i0b, i1b, _, _, u0, u1, _, _, sem, _, isem = sets[s]
                base = _idx_base(j)
                pltpu.make_async_copy(i0h.at[pl.ds(base, ch)], i0b, isem).wait()
                pltpu.make_async_copy(i1h.at[pl.ds(base, ch)], i1b, isem).wait()
                pltpu.async_copy(ta0.at[i0b], u0, sem)
                pltpu.async_copy(ta1.at[i1b], u1, sem)

            def _save_idx(s):
                i0b, i1b, c0b, c1b = sets[s][0], sets[s][1], sets[s][2], sets[s][3]
                for off in (0, 16, ch - 16):
                    sl = pl.ds(off, 16)
                    c0b[sl] = i0b[sl]
                    c1b[sl] = i1b[sl]

            def _wait_gathers(s):
                i0b, i1b, _, _, u0, u1, _, _, sem, _, _ = sets[s]
                pltpu.make_async_copy(ta0.at[i0b], u0, sem).wait()
                pltpu.make_async_copy(ta1.at[i1b], u1, sem).wait()

            def _consume(s):
                _, _, c0b, c1b, u0, u1, h0, h1, sem, ssem, _ = sets[s]

                def _row(r2, _):
                    for sub in (0, 1):
                        r = 2 * r2 + sub
                        for half, hb in ((0, h0), (1, h1)):
                            for k in range(d // 32):
                                w0 = u0[r, pl.ds(half * (d // 2) + k * 16, 16)]
                                w1 = u1[r, pl.ds(half * (d // 2) + k * 16, 16)]
                                lo0 = lax.bitcast_convert_type(w0 << 16, _F32)
                                lo1 = lax.bitcast_convert_type(w1 << 16, _F32)
                                hi0 = lax.bitcast_convert_type(
                                    w0 & jnp.int32(-65536), _F32)
                                hi1 = lax.bitcast_convert_type(
                                    w1 & jnp.int32(-65536), _F32)
                                hb[r, pl.ds(k * 32, 16)] = jnp.maximum(
                                    lo0 + lo1, 0.0)
                                hb[r, pl.ds(k * 32 + 16, 16)] = jnp.maximum(
                                    hi0 + hi1, 0.0)
                    return 0
                lax.fori_loop(0, ch // 2, _row, 0)

                pltpu.async_copy(h0, accum.at[c0b], ssem, add=True)
                pltpu.async_copy(h1, accum.at[c1b], ssem, add=True)

            def _wait_scatter(s):
                _, _, c0b, c1b, _, _, h0, h1, _, ssem, _ = sets[s]
                pltpu.make_async_copy(h0, accum.at[c0b], ssem).wait()
                pltpu.make_async_copy(h1, accum.at[c1b], ssem).wait()

            _prefetch_idx(0, 0)
            _prefetch_idx(1, 1)
            _issue(0, 0)
            _issue(1, 1)

            def _pair(p, _):
                @pl.when(p > 0)
                def _():
                    _wait_scatter(0)
                    _wait_scatter(1)
                _wait_gathers(0)
                _save_idx(0)
                _prefetch_idx(0, 2 * p + 2)
                _consume(0)
                _issue(0, 2 * p + 2)

                _wait_gathers(1)
                _save_idx(1)

                @pl.when(p < npairs - 1)
                def _():
                    _prefetch_idx(1, 2 * p + 3)
                _consume(1)

                @pl.when(p < npairs - 1)
                def _():
                    _issue(1, 2 * p + 3)
                return 0
            lax.fori_loop(0, npairs, _pair, 0)
            _wait_scatter(0)
            _wait_scatter(1)
            _wait_gathers(0)
            _save_idx(0)
            _consume(0)
            _wait_scatter(0)

            plsc.subcore_barrier()
            pltpu.sync_copy(accum.at[pl.ds(row0, rows_per_tile)],
                            acc_out.at[pl.ds(cid * npad + row0, rows_per_tile)])

        _run_pass(tfa0, tfa1, ia0h, ia1h, acc_out_a)

    return pl.kernel(body, mesh=mesh, out_type=out_type, scratch_types=scratch), npad


def kernel(inputs, grounding_indices_0, grounding_indices_1,
           W_f0_t0, b_f0_t0, W_f0_t1, b_f0_t1,
           W_f1_t0, b_f1_t0, W_f1_t1, b_f1_t1,
           attention_weights):
    x = inputs[0]
    n, d = x.shape
    g = grounding_indices_0.shape[0]

    ia0 = grounding_indices_0[:, 0].astype(jnp.int32)
    ia1 = grounding_indices_0[:, 1].astype(jnp.int32)
    ib0 = grounding_indices_1[:, 0].astype(jnp.int32)
    ib1 = grounding_indices_1[:, 1].astype(jnp.int32)

    ba0 = b_f0_t0.reshape(1, -1)
    ba1 = b_f0_t1.reshape(1, -1)
    bb0 = b_f1_t0.reshape(1, -1)
    bb1 = b_f1_t1.reshape(1, -1)

    sc_pass, npad = _make_sc_pass(n, d, g)
    sc_count = _make_sc_count(n, d, g)

    cnta = sc_count(ia0, ia1)[0].reshape(2, npad, d)[:, :n, 0:16]
    cntb = sc_count(ib0, ib1)[0].reshape(2, npad, d)[:, :n, 0:16]

    tabs = _mm_layer0(x, W_f0_t0, ba0, W_f1_t0, bb0)
    acca = sc_pass(tabs[0], tabs[1], ia0, ia1)[0]
    accb = sc_pass(tabs[2], tabs[3], ib0, ib1)[0]
    acca = acca.reshape(2, npad, d)[:, :n]
    accb = accb.reshape(2, npad, d)[:, :n]

    tabs1 = _mm_layer1(acca, cnta, accb, cntb, W_f0_t1, ba1, W_f1_t1, bb1)
    acca1 = sc_pass(tabs1[0], tabs1[1], ia0, ia1)[0]
    accb1 = sc_pass(tabs1[2], tabs1[3], ib0, ib1)[0]
    acca1 = acca1.reshape(2, npad, d)[:, :n]
    accb1 = accb1.reshape(2, npad, d)[:, :n]

    out = _merge_final(acca1, cnta, accb1, cntb)
    return out[None]
